# Initial kernel scaffold; baseline (speedup 1.0000x reference)
import jax
import jax.numpy as jnp
from jax.experimental import pallas as pl


def kernel(x, batch):
    raise NotImplementedError("write your pallas kernel here")


# speedup vs baseline: 35.4462x; 35.4462x over previous
import functools

import jax
import jax.numpy as jnp
from jax import lax
from jax.experimental import pallas as pl
from jax.experimental.pallas import tpu as pltpu
from jax.experimental.pallas import tpu_sc as plsc

N = 100000
NPAD = 131072
ROWS = 1024
LANES = 128
B = 64
K = 1024
NSLOTS = B * K


def _sort_body(keys_ref, batch_ref, gidx_ref, perm_ref, mask_ref):
    keys = keys_ref[...]
    batch = batch_ref[...]

    rows = lax.broadcasted_iota(jnp.int32, (ROWS, LANES), 0)
    lanes = lax.broadcasted_iota(jnp.int32, (ROWS, LANES), 1)

    kb = lax.bitcast_convert_type(keys, jnp.uint32)
    asc = jnp.where(kb >= jnp.uint32(0x80000000), ~kb, kb | jnp.uint32(0x80000000))
    ku = ~asc

    idxu = (rows * 128 + lanes).astype(jnp.uint32)
    bu = batch.astype(jnp.uint32)
    w1u = (bu << 26) | (ku >> 6)
    w2u = ((ku & jnp.uint32(63)) << 17) | idxu
    pad = batch >= B
    big = jnp.int32(0x7FFFFFFF)
    w1 = jnp.where(pad, big, lax.bitcast_convert_type(w1u ^ jnp.uint32(0x80000000), jnp.int32))
    w2 = jnp.where(pad, big, lax.bitcast_convert_type(w2u ^ jnp.uint32(0x80000000), jnp.int32))

    for ke in range(1, 18):
        kbit = 1 << ke
        for je in range(ke - 1, -1, -1):
            j = 1 << je
            if j >= LANES:
                r = j // LANES
                lower = (rows & r) == 0
                b1 = jnp.where(lower, jnp.roll(w1, -r, axis=0), jnp.roll(w1, r, axis=0))
                b2 = jnp.where(lower, jnp.roll(w2, -r, axis=0), jnp.roll(w2, r, axis=0))
            else:
                lower = (lanes & j) == 0
                b1 = jnp.where(lower, jnp.roll(w1, -j, axis=1), jnp.roll(w1, j, axis=1))
                b2 = jnp.where(lower, jnp.roll(w2, -j, axis=1), jnp.roll(w2, j, axis=1))
            if kbit >= NPAD:
                up = jnp.full((ROWS, LANES), True)
            elif kbit >= LANES:
                up = (rows & (kbit // LANES)) == 0
            else:
                up = (lanes & kbit) == 0
            a_gt_b = (w1 > b1) | ((w1 == b1) & (w2 > b2))
            cond = lower == up
            take_b = a_gt_b ^ (~cond)
            w1 = jnp.where(take_b, b1, w1)
            w2 = jnp.where(take_b, b2, w2)

    cnts = [jnp.sum((batch == g).astype(jnp.int32)) for g in range(B)]
    ptrs = []
    run = jnp.int32(0)
    for g in range(B):
        ptrs.append(run)
        run = run + cnts[g]
    max_nodes = functools.reduce(jnp.maximum, cnts)

    w2u_s = lax.bitcast_convert_type(w2, jnp.uint32) ^ jnp.uint32(0x80000000)
    idxf = (w2u_s & jnp.uint32(0x1FFFF)).astype(jnp.int32).astype(jnp.float32)

    j8 = (lax.broadcasted_iota(jnp.int32, (8, LANES), 0) * 128
          + lax.broadcasted_iota(jnp.int32, (8, LANES), 1))
    s16 = lax.broadcasted_iota(jnp.int32, (16, ROWS), 0)
    row16 = lax.broadcasted_iota(jnp.int32, (16, ROWS), 1)
    ci = lax.broadcasted_iota(jnp.int32, (LANES, LANES), 0)
    co = lax.broadcasted_iota(jnp.int32, (LANES, LANES), 1)

    for g in range(B):
        p = ptrs[g]
        r0 = p // 128
        o = p % 128
        sel = (row16 == s16 + r0).astype(jnp.float32)
        r16 = jnp.dot(sel, idxf, preferred_element_type=jnp.float32,
                      precision=lax.Precision.HIGHEST)
        a = r16[0:8]
        bm = r16[1:9]
        sl = (ci == co + o).astype(jnp.float32)
        sr = (ci + 128 == co + o).astype(jnp.float32)
        out8f = (jnp.dot(a, sl, preferred_element_type=jnp.float32,
                         precision=lax.Precision.HIGHEST)
                 + jnp.dot(bm, sr, preferred_element_type=jnp.float32,
                           precision=lax.Precision.HIGHEST))
        gidx8 = out8f.astype(jnp.int32)

        num_g = cnts[g]
        validn = j8 < num_g
        ploc = gidx8 - p
        perm8 = jnp.where(j8 < max_nodes, jnp.where(validn, ploc, j8), -1)
        safe8 = jnp.where(validn, gidx8, jnp.int32(N))
        gidx_ref[g] = safe8
        perm_ref[g] = perm8
        mask_ref[g] = validn.astype(jnp.int32)


def _run_sort(keys2d, batch2d):
    return pl.pallas_call(
        _sort_body,
        out_shape=[
            jax.ShapeDtypeStruct((B, 8, LANES), jnp.int32),
            jax.ShapeDtypeStruct((B, 8, LANES), jnp.int32),
            jax.ShapeDtypeStruct((B, 8, LANES), jnp.int32),
        ],
    )(keys2d, batch2d)


def _sc_gather(table, idx):
    nc, ns = 2, 16
    per_w = NSLOTS // (nc * ns)
    chunk = 256
    mesh = plsc.VectorSubcoreMesh(core_axis_name="c", subcore_axis_name="s")

    @functools.partial(
        pl.kernel,
        mesh=mesh,
        out_type=jax.ShapeDtypeStruct((NSLOTS, LANES), jnp.float32),
        scratch_types=[
            pltpu.VMEM((chunk,), jnp.int32),
            pltpu.VMEM((chunk, LANES), jnp.float32),
            pltpu.SemaphoreType.DMA,
        ],
    )
    def k(table_hbm, idx_hbm, out_hbm, idx_v, rows_v, sem):
        wid = lax.axis_index("s") * nc + lax.axis_index("c")
        base = wid * per_w
        for c in range(per_w // chunk):
            b = base + c * chunk
            pltpu.sync_copy(idx_hbm.at[pl.ds(b, chunk)], idx_v)
            pltpu.async_copy(table_hbm.at[idx_v], rows_v, sem).wait()
            pltpu.sync_copy(rows_v, out_hbm.at[pl.ds(b, chunk)])

    return k(table, idx)


def kernel(x, batch):
    keys = jnp.pad(x[:, LANES - 1], (0, NPAD - N))
    batch_p = jnp.pad(batch.astype(jnp.int32), (0, NPAD - N), constant_values=B)
    gidx3, perm3, mask3 = _run_sort(keys.reshape(ROWS, LANES),
                                    batch_p.reshape(ROWS, LANES))

    xp = jnp.concatenate([x, jnp.zeros((1, LANES), jnp.float32)], axis=0)
    pooled = _sc_gather(xp, gidx3.reshape(NSLOTS))

    pooled_x = pooled.reshape(B, K, LANES)
    pooled_perm = perm3.reshape(B, K)
    pooled_mask = mask3.reshape(B, K).astype(bool)
    return pooled_x, pooled_perm, pooled_mask
